# baseline (speedup 1.0000x reference)
import jax
import jax.numpy as jnp
import numpy as np
from jax import lax
from jax.experimental import pallas as pl
from jax.experimental.pallas import tpu as pltpu
from jax.experimental.pallas import tpu_sc as plsc

N_NODES, D = 50000, 256
N0, N1, N2, S = 262144, 16384, 1024, 16
NC, NS = 2, 16
NW = NC * NS
DW = D // 2
_HI = np.uint32(0xFFFF0000)


def _split(w):
    u = plsc.bitcast(w, jnp.uint32)
    return (plsc.bitcast(u << 16, jnp.float32),
            plsc.bitcast(u & _HI, jnp.float32))
LANES = 16
NB = 8
CHUNK = NB * S
WAVE = 16


def _make_agg(n_out, compose):
    rows_w = n_out // NW
    iters = rows_w // NB
    mesh = plsc.VectorSubcoreMesh(core_axis_name="c", subcore_axis_name="s")
    scale = 1.0 / S

    def body(nodes0_hbm, sidx_hbm, nidx_hbm, tab_hbm, selfv_hbm, neigh_hbm,
             nidx_v, sidx_v, sg_v, g_all, rows_v, srows_v, acc_v,
             sem_i, sem_g, sem_s, sem_oa, sem_os):
        wid = lax.axis_index("s") * NC + lax.axis_index("c")
        base = wid * rows_w
        pltpu.sync_copy(nidx_hbm.at[pl.ds(base * S, rows_w * S)], nidx_v)
        pltpu.sync_copy(sidx_hbm.at[pl.ds(base, rows_w)], sidx_v)

        if compose:
            n_chunks = rows_w * S // CHUNK

            def wave_body(w, _):
                def fire(j, _):
                    k = w * WAVE + j
                    pltpu.async_copy(
                        nodes0_hbm.at[nidx_v.at[pl.ds(k * CHUNK, CHUNK)]],
                        g_all.at[pl.ds(k * CHUNK, CHUNK)], sem_i)
                    return 0
                lax.fori_loop(0, WAVE, fire, 0)
                def drain(j, _):
                    pltpu.make_async_copy(
                        nodes0_hbm.at[nidx_v.at[pl.ds(j * CHUNK, CHUNK)]],
                        g_all.at[pl.ds(j * CHUNK, CHUNK)], sem_i).wait()
                    return 0
                lax.fori_loop(0, WAVE, drain, 0)
                return 0
            lax.fori_loop(0, max(n_chunks // WAVE, 1), wave_body, 0)
            for j in range(0, rows_w, CHUNK):
                n = min(CHUNK, rows_w - j)
                pltpu.async_copy(nodes0_hbm.at[sidx_v.at[pl.ds(j, n)]],
                                 sg_v.at[pl.ds(j, n)], sem_i).wait()
            g_src, s_src = g_all, sg_v
        else:
            g_src, s_src = nidx_v, sidx_v

        def rows_cp(i, b):
            return pltpu.make_async_copy(
                tab_hbm.at[g_src.at[pl.ds(i * CHUNK, CHUNK)]],
                rows_v.at[b], sem_g)

        def self_cp(i, b4):
            return pltpu.make_async_copy(
                tab_hbm.at[s_src.at[pl.ds(i * NB, NB)]],
                srows_v.at[b4], sem_s)

        def out_acc_cp(i, b4):
            return pltpu.make_async_copy(
                acc_v.at[b4], neigh_hbm.at[pl.ds(base + i * NB, NB), :], sem_oa)

        def out_self_cp(i, b4):
            return pltpu.make_async_copy(
                srows_v.at[b4], selfv_hbm.at[pl.ds(base + i * NB, NB), :], sem_os)

        def accumulate(b2, b4):
            def grp(gi, _):
                r0 = gi * S
                for c in range(DW // LANES):
                    cw = c * LANES
                    ae, ao = _split(rows_v[b2, r0, pl.ds(cw, LANES)])
                    for r in range(1, S):
                        ve, vo = _split(rows_v[b2, r0 + r, pl.ds(cw, LANES)])
                        ae = ae + ve
                        ao = ao + vo
                    p = plsc.pack(ae * scale, ao * scale,
                                  format=plsc.PackFormat.INTERLEAVED)
                    acc_v[b4, gi, pl.ds(cw, LANES)] = plsc.bitcast(p, jnp.int32)
                return 0
            lax.fori_loop(0, NB, grp, 0)

        rows_cp(0, 0).start()
        self_cp(0, 0).start()
        rows_cp(1, 1).start()
        self_cp(1, 1).start()

        def outer(o, _):
            i0 = o * 4
            for u in range(4):
                i = i0 + u
                b2 = u % 2
                @pl.when(i >= 2)
                def _():
                    out_acc_cp(i - 2, (u + 2) % 4).wait()
                    out_self_cp(i - 2, (u + 2) % 4).wait()
                rows_cp(i, b2).wait()
                self_cp(i, u).wait()
                accumulate(b2, u)
                out_acc_cp(i, u).start()
                out_self_cp(i, u).start()
                @pl.when(i + 2 < iters)
                def _():
                    rows_cp(i + 2, b2).start()
                    self_cp(i + 2, (u + 2) % 4).start()
            return 0

        lax.fori_loop(0, iters // 4, outer, 0)
        out_acc_cp(iters - 2, (iters - 2) % 4).wait()
        out_self_cp(iters - 2, (iters - 2) % 4).wait()
        out_acc_cp(iters - 1, (iters - 1) % 4).wait()
        out_self_cp(iters - 1, (iters - 1) % 4).wait()

    i32 = jnp.int32
    return pl.kernel(
        body,
        out_type=(jax.ShapeDtypeStruct((n_out, DW), i32),
                  jax.ShapeDtypeStruct((n_out, DW), i32)),
        mesh=mesh,
        compiler_params=pltpu.CompilerParams(needs_layout_passes=False),
        scratch_types=[
            pltpu.VMEM((rows_w * S,), i32),
            pltpu.VMEM((rows_w,), i32),
            pltpu.VMEM((rows_w,), i32),
            pltpu.VMEM((rows_w * S,), i32),
            pltpu.VMEM((2, CHUNK, DW), i32),
            pltpu.VMEM((4, NB, DW), i32),
            pltpu.VMEM((4, NB, DW), i32),
            pltpu.SemaphoreType.DMA,
            pltpu.SemaphoreType.DMA,
            pltpu.SemaphoreType.DMA,
            pltpu.SemaphoreType.DMA,
            pltpu.SemaphoreType.DMA,
        ],
    )


def _make_mm_body(out_dtype):
    def _mm_body(sv_ref, nb_ref, wa_ref, wb_ref, b_ref, o_ref):
        acc = jnp.dot(sv_ref[...], wa_ref[...],
                      preferred_element_type=jnp.float32)
        acc = acc + jnp.dot(nb_ref[...], wb_ref[...],
                            preferred_element_type=jnp.float32)
        o_ref[...] = jnp.maximum(acc + b_ref[...], 0.0).astype(out_dtype)
    return _mm_body


def _encoder(selfv, neigh, W, b, bm, out_dtype):
    n = selfv.shape[0]
    Wb = W.astype(jnp.bfloat16)
    return pl.pallas_call(
        _make_mm_body(out_dtype),
        grid=(n // bm,),
        in_specs=[
            pl.BlockSpec((bm, D), lambda i: (i, 0)),
            pl.BlockSpec((bm, D), lambda i: (i, 0)),
            pl.BlockSpec((D, D), lambda i: (0, 0)),
            pl.BlockSpec((D, D), lambda i: (0, 0)),
            pl.BlockSpec((1, D), lambda i: (0, 0)),
        ],
        out_specs=pl.BlockSpec((bm, D), lambda i: (i, 0)),
        out_shape=jax.ShapeDtypeStruct((n, D), out_dtype),
    )(selfv, neigh, Wb[:D], Wb[D:], b.reshape(1, D))


_agg1 = _make_agg(N1, compose=True)
_agg2 = _make_agg(N2, compose=False)


def _as_words(x16):
    n = x16.shape[0]
    return lax.bitcast_convert_type(x16.reshape(n, DW, 2), jnp.int32)


def _as_bf16(xw):
    n = xw.shape[0]
    return lax.bitcast_convert_type(xw, jnp.bfloat16).reshape(n, D)


@jax.jit
def kernel(nodes0, self_idx1, neigh1, self_idx2, neigh2, table, W1, b1, W2, b2):
    nodes0 = nodes0.astype(jnp.int32)
    tabw = _as_words(table.astype(jnp.bfloat16))
    sw1, nw1 = _agg1(nodes0, self_idx1.astype(jnp.int32),
                     neigh1.astype(jnp.int32).reshape(-1), tabw)
    h1 = _encoder(_as_bf16(sw1), _as_bf16(nw1), W1, b1, 2048, jnp.bfloat16)
    sw2, nw2 = _agg2(nodes0, self_idx2.astype(jnp.int32),
                     neigh2.astype(jnp.int32).reshape(-1), _as_words(h1))
    h2 = _encoder(_as_bf16(sw2), _as_bf16(nw2), W2, b2, 1024, jnp.float32)
    return h2


# speedup vs baseline: 3.9751x; 3.9751x over previous
import jax
import jax.numpy as jnp
import numpy as np
from jax import lax
from jax.experimental import pallas as pl
from jax.experimental.pallas import tpu as pltpu
from jax.experimental.pallas import tpu_sc as plsc

N_NODES, D = 50000, 256
N0, N1, N2, S = 262144, 16384, 1024, 16
NC, NS = 2, 16
NW = NC * NS
LANES = 16
DW = D // 2
_HI = np.uint32(0xFFFF0000)


def _split(w):
    u = plsc.bitcast(w, jnp.uint32)
    return (plsc.bitcast(u << 16, jnp.float32),
            plsc.bitcast(u & _HI, jnp.float32))
NB = 8
CHUNK = NB * S
WAVE = 16


def _make_agg(n_out, compose):
    rows_w = n_out // NW
    iters = rows_w // NB
    mesh = plsc.VectorSubcoreMesh(core_axis_name="c", subcore_axis_name="s")
    scale = 1.0 / S

    def body(nodes0_hbm, sidx_hbm, nidx_hbm, tab_hbm, selfv_hbm, neigh_hbm,
             nidx_v, sidx_v, sg_v, g_all, rows_v, srows_v, acc_v,
             sem_i, sem_g, sem_s, sem_oa, sem_os):
        wid = lax.axis_index("s") * NC + lax.axis_index("c")
        base = wid * rows_w
        pltpu.sync_copy(nidx_hbm.at[pl.ds(base * S, rows_w * S)], nidx_v)
        pltpu.sync_copy(sidx_hbm.at[pl.ds(base, rows_w)], sidx_v)

        if compose:
            n_chunks = rows_w * S // CHUNK

            def wave_body(w, _):
                def fire(j, _):
                    k = w * WAVE + j
                    pltpu.async_copy(
                        nodes0_hbm.at[nidx_v.at[pl.ds(k * CHUNK, CHUNK)]],
                        g_all.at[pl.ds(k * CHUNK, CHUNK)], sem_i)
                    return 0
                lax.fori_loop(0, WAVE, fire, 0)
                def drain(j, _):
                    pltpu.make_async_copy(
                        nodes0_hbm.at[nidx_v.at[pl.ds(j * CHUNK, CHUNK)]],
                        g_all.at[pl.ds(j * CHUNK, CHUNK)], sem_i).wait()
                    return 0
                lax.fori_loop(0, WAVE, drain, 0)
                return 0
            lax.fori_loop(0, max(n_chunks // WAVE, 1), wave_body, 0)
            for j in range(0, rows_w, CHUNK):
                n = min(CHUNK, rows_w - j)
                pltpu.async_copy(nodes0_hbm.at[sidx_v.at[pl.ds(j, n)]],
                                 sg_v.at[pl.ds(j, n)], sem_i).wait()
            g_src, s_src = g_all, sg_v
        else:
            g_src, s_src = nidx_v, sidx_v

        def rows_cp(i, b):
            return pltpu.make_async_copy(
                tab_hbm.at[g_src.at[pl.ds(i * CHUNK, CHUNK)]],
                rows_v.at[b], sem_g)

        def self_cp(i, b4):
            return pltpu.make_async_copy(
                tab_hbm.at[s_src.at[pl.ds(i * NB, NB)]],
                srows_v.at[b4], sem_s)

        def out_acc_cp(i, b4):
            return pltpu.make_async_copy(
                acc_v.at[b4], neigh_hbm.at[pl.ds(base + i * NB, NB), :], sem_oa)

        def out_self_cp(i, b4):
            return pltpu.make_async_copy(
                srows_v.at[b4], selfv_hbm.at[pl.ds(base + i * NB, NB), :], sem_os)

        def accumulate(b2, b4):
            def grp(gi, _):
                r0 = gi * S
                for c in range(DW // LANES):
                    cw = c * LANES
                    ae, ao = _split(rows_v[b2, r0, pl.ds(cw, LANES)])
                    for r in range(1, S):
                        ve, vo = _split(rows_v[b2, r0 + r, pl.ds(cw, LANES)])
                        ae = ae + ve
                        ao = ao + vo
                    p = plsc.pack(ae * scale, ao * scale,
                                  format=plsc.PackFormat.INTERLEAVED)
                    acc_v[b4, gi, pl.ds(cw, LANES)] = plsc.bitcast(p, jnp.int32)
                return 0
            lax.fori_loop(0, NB, grp, 0)

        rows_cp(0, 0).start()
        self_cp(0, 0).start()
        rows_cp(1, 1).start()
        self_cp(1, 1).start()

        def outer(o, _):
            i0 = o * 4
            for u in range(4):
                i = i0 + u
                b2 = u % 2
                @pl.when(i >= 2)
                def _():
                    out_acc_cp(i - 2, (u + 2) % 4).wait()
                    out_self_cp(i - 2, (u + 2) % 4).wait()
                rows_cp(i, b2).wait()
                self_cp(i, u).wait()
                accumulate(b2, u)
                out_acc_cp(i, u).start()
                out_self_cp(i, u).start()
                @pl.when(i + 2 < iters)
                def _():
                    rows_cp(i + 2, b2).start()
                    self_cp(i + 2, (u + 2) % 4).start()
            return 0

        lax.fori_loop(0, iters // 4, outer, 0)
        out_acc_cp(iters - 2, (iters - 2) % 4).wait()
        out_self_cp(iters - 2, (iters - 2) % 4).wait()
        out_acc_cp(iters - 1, (iters - 1) % 4).wait()
        out_self_cp(iters - 1, (iters - 1) % 4).wait()

    i32 = jnp.int32
    return pl.kernel(
        body,
        out_type=(jax.ShapeDtypeStruct((n_out, DW), i32),
                  jax.ShapeDtypeStruct((n_out, DW), i32)),
        mesh=mesh,
        compiler_params=pltpu.CompilerParams(needs_layout_passes=False),
        scratch_types=[
            pltpu.VMEM((rows_w * S,), i32),
            pltpu.VMEM((rows_w,), i32),
            pltpu.VMEM((rows_w,), i32),
            pltpu.VMEM((rows_w * S,), i32),
            pltpu.VMEM((2, CHUNK, DW), i32),
            pltpu.VMEM((4, NB, DW), i32),
            pltpu.VMEM((4, NB, DW), i32),
            pltpu.SemaphoreType.DMA,
            pltpu.SemaphoreType.DMA,
            pltpu.SemaphoreType.DMA,
            pltpu.SemaphoreType.DMA,
            pltpu.SemaphoreType.DMA,
        ],
    )


def _pack_words(l, r):
    bl = lax.bitcast_convert_type(l.astype(jnp.bfloat16), jnp.int16)
    br = lax.bitcast_convert_type(r.astype(jnp.bfloat16), jnp.int16)
    return (bl.astype(jnp.int32) & 0xFFFF) | (br.astype(jnp.int32) << 16)


def _unpack_words(w):
    lo = lax.bitcast_convert_type(w << 16, jnp.float32)
    hi = lax.bitcast_convert_type(w & -65536, jnp.float32)
    return lo, hi


def _make_mm_body(words_out):
    def _mm_body(sv_ref, nb_ref, wsl_ref, wsh_ref, wnl_ref, wnh_ref, b_ref,
                 o_ref):
        svl, svh = _unpack_words(sv_ref[...])
        nbl, nbh = _unpack_words(nb_ref[...])
        acc = jnp.dot(svl.astype(jnp.bfloat16),
                      wsl_ref[...].astype(jnp.bfloat16),
                      preferred_element_type=jnp.float32)
        acc += jnp.dot(svh.astype(jnp.bfloat16),
                       wsh_ref[...].astype(jnp.bfloat16),
                       preferred_element_type=jnp.float32)
        acc += jnp.dot(nbl.astype(jnp.bfloat16),
                       wnl_ref[...].astype(jnp.bfloat16),
                       preferred_element_type=jnp.float32)
        acc += jnp.dot(nbh.astype(jnp.bfloat16),
                       wnh_ref[...].astype(jnp.bfloat16),
                       preferred_element_type=jnp.float32)
        h = jnp.maximum(acc + b_ref[...], 0.0)
        if words_out:
            o_ref[...] = _pack_words(h[:, :DW], h[:, DW:])
        else:
            o_ref[...] = h
    return _mm_body


def _encoder(selfw, neighw, W, b, bm, words_out):
    n = selfw.shape[0]
    od = DW if words_out else D
    ot = jnp.int32 if words_out else jnp.float32
    wspec = pl.BlockSpec((DW, D), lambda i: (0, 0))
    return pl.pallas_call(
        _make_mm_body(words_out),
        grid=(n // bm,),
        in_specs=[
            pl.BlockSpec((bm, DW), lambda i: (i, 0)),
            pl.BlockSpec((bm, DW), lambda i: (i, 0)),
            wspec, wspec, wspec, wspec,
            pl.BlockSpec((1, D), lambda i: (0, 0)),
        ],
        out_specs=pl.BlockSpec((bm, od), lambda i: (i, 0)),
        out_shape=jax.ShapeDtypeStruct((n, od), ot),
    )(selfw, neighw, W[:DW], W[DW:D], W[D:D + DW], W[D + DW:], b.reshape(1, D))


_agg1 = _make_agg(N1, compose=True)
_agg2 = _make_agg(N2, compose=False)


def _cast_body(x_ref, o_ref):
    x = x_ref[...]
    o_ref[...] = _pack_words(x[:, :DW], x[:, DW:])


def _to_words(x, bm):
    n = x.shape[0]
    return pl.pallas_call(
        _cast_body,
        grid=(n // bm,),
        in_specs=[pl.BlockSpec((bm, D), lambda i: (i, 0))],
        out_specs=pl.BlockSpec((bm, DW), lambda i: (i, 0)),
        out_shape=jax.ShapeDtypeStruct((n, DW), jnp.int32),
    )(x)


@jax.jit
def kernel(nodes0, self_idx1, neigh1, self_idx2, neigh2, table, W1, b1, W2, b2):
    nodes0 = nodes0.astype(jnp.int32)
    tabw = _to_words(table, 2000)
    sw1, nw1 = _agg1(nodes0, self_idx1.astype(jnp.int32),
                     neigh1.astype(jnp.int32).reshape(-1), tabw)
    h1w = _encoder(sw1, nw1, W1, b1, 2048, True)
    sw2, nw2 = _agg2(nodes0, self_idx2.astype(jnp.int32),
                     neigh2.astype(jnp.int32).reshape(-1), h1w)
    h2 = _encoder(sw2, nw2, W2, b2, 1024, False)
    return h2
